# baseline (device time: 7874 ns/iter reference)
import jax
import jax.numpy as jnp
from jax import lax
from jax.experimental import pallas as pl
from jax.experimental.pallas import tpu as pltpu

N_X = 2


def kernel(x):
    m_per, n = x.shape

    def body(x_hbm, out_hbm, x_vmem, mine_vmem, in_sem, out_sem,
             send_sem, recv_sem):
        my_x = lax.axis_index("x")
        my_y = lax.axis_index("y")
        nbr = (1 - my_x, my_y)

        load = pltpu.make_async_copy(x_hbm, x_vmem, in_sem)
        load.start()

        barrier_sem = pltpu.get_barrier_semaphore()
        pl.semaphore_signal(
            barrier_sem, inc=1,
            device_id=nbr, device_id_type=pltpu.DeviceIdType.MESH,
        )
        pl.semaphore_wait(barrier_sem, 1)

        load.wait()
        mine_vmem[...] = x_vmem[...].astype(jnp.bfloat16)

        rdma = pltpu.make_async_remote_copy(
            src_ref=mine_vmem,
            dst_ref=out_hbm.at[pl.ds(my_x * m_per, m_per)],
            send_sem=send_sem,
            recv_sem=recv_sem,
            device_id=nbr,
            device_id_type=pltpu.DeviceIdType.MESH,
        )
        rdma.start()

        store = pltpu.make_async_copy(
            mine_vmem, out_hbm.at[pl.ds(my_x * m_per, m_per)], out_sem
        )
        store.start()
        store.wait()
        rdma.wait()

    return pl.pallas_call(
        body,
        out_shape=pltpu.MemorySpace.HBM((N_X * m_per, n), jnp.bfloat16),
        in_specs=[pl.BlockSpec(memory_space=pltpu.MemorySpace.HBM)],
        out_specs=pl.BlockSpec(memory_space=pltpu.MemorySpace.HBM),
        scratch_shapes=[
            pltpu.VMEM((m_per, n), jnp.float32),
            pltpu.VMEM((m_per, n), jnp.bfloat16),
            pltpu.SemaphoreType.DMA,
            pltpu.SemaphoreType.DMA,
            pltpu.SemaphoreType.DMA,
            pltpu.SemaphoreType.DMA,
        ],
        compiler_params=pltpu.CompilerParams(collective_id=0),
    )(pltpu.with_memory_space_constraint(x, pltpu.MemorySpace.HBM))


# device time: 6719 ns/iter; 1.1719x vs baseline; 1.1719x over previous
import jax
import jax.numpy as jnp
from jax import lax
from jax.experimental import pallas as pl
from jax.experimental.pallas import tpu as pltpu

N_X = 2


def kernel(x):
    m_per, n = x.shape

    def body(x_hbm, out_ref, x_vmem, in_sem, send_sem, recv_sem):
        my_x = lax.axis_index("x")
        my_y = lax.axis_index("y")
        nbr = (1 - my_x, my_y)

        load = pltpu.make_async_copy(x_hbm, x_vmem, in_sem)
        load.start()

        barrier_sem = pltpu.get_barrier_semaphore()
        pl.semaphore_signal(
            barrier_sem, inc=1,
            device_id=nbr, device_id_type=pltpu.DeviceIdType.MESH,
        )
        pl.semaphore_wait(barrier_sem, 1)

        load.wait()
        out_ref[pl.ds(my_x * m_per, m_per), :] = x_vmem[...].astype(
            jnp.bfloat16
        )

        rdma = pltpu.make_async_remote_copy(
            src_ref=out_ref.at[pl.ds(my_x * m_per, m_per)],
            dst_ref=out_ref.at[pl.ds(my_x * m_per, m_per)],
            send_sem=send_sem,
            recv_sem=recv_sem,
            device_id=nbr,
            device_id_type=pltpu.DeviceIdType.MESH,
        )
        rdma.start()
        rdma.wait()

    return pl.pallas_call(
        body,
        out_shape=jax.ShapeDtypeStruct((N_X * m_per, n), jnp.bfloat16),
        in_specs=[pl.BlockSpec(memory_space=pltpu.MemorySpace.HBM)],
        out_specs=pl.BlockSpec(memory_space=pltpu.MemorySpace.VMEM),
        scratch_shapes=[
            pltpu.VMEM((m_per, n), jnp.float32),
            pltpu.SemaphoreType.DMA,
            pltpu.SemaphoreType.DMA,
            pltpu.SemaphoreType.DMA,
        ],
        compiler_params=pltpu.CompilerParams(collective_id=0),
    )(pltpu.with_memory_space_constraint(x, pltpu.MemorySpace.HBM))
